# baseline (speedup 1.0000x reference)
import functools

import jax
import jax.numpy as jnp
from jax import lax
from jax.experimental import pallas as pl
from jax.experimental.pallas import tpu as pltpu
from jax.experimental.pallas import tpu_sc as plsc

N = 10000
NP = 10240
E = 320000
NC, NS = 2, 16
NW = NC * NS
CHUNK = E // NW
B = 80
EP = 10000
NBLK = EP // B
STRIPE = NP // NS
SRCHUNK = 128
NZC = STRIPE // SRCHUNK
D = 32
NQ = 4
CW = 16
KBUF = 5
PF = 3


def _sc_body(xs0, xs1, xs2, xs3, xm, srcs_h, dsts_h, ews_h, srcm_h, dstm_h, ewm_h,
             Ss0, Ss1, Ss2, Ss3, Smp, csp, cmp,
             srcv, dstv, ewv, rowbuf, rowC, zbS, zbc, wbS, wbc,
             gsem, ssem, csem, zsem, wsem,
             S_sh, c_sh):
    cid = lax.axis_index("c")
    sid = lax.axis_index("s")
    wid = cid * NS + sid
    z16 = jnp.zeros((16,), jnp.float32)

    def zrow(r, carry):
        for k in range(D // 16):
            zbS[r, pl.ds(k * 16, 16)] = z16
        zbc[r, :] = z16
        return carry

    lax.fori_loop(0, SRCHUNK, zrow, 0)

    def run_phase(x_hbm, Sp, cp, with_c):
        for j in range(NZC):
            sl = pl.ds(sid * STRIPE + j * SRCHUNK, SRCHUNK)
            pltpu.async_copy(zbS, S_sh.at[sl], zsem)
            if with_c:
                pltpu.async_copy(zbc, c_sh.at[sl], zsem)
        for j in range(NZC):
            sl = pl.ds(sid * STRIPE + j * SRCHUNK, SRCHUNK)
            pltpu.make_async_copy(zbS, S_sh.at[sl], zsem).wait()
            if with_c:
                pltpu.make_async_copy(zbc, c_sh.at[sl], zsem).wait()
        plsc.subcore_barrier()

        def gather(blk, b):
            pltpu.async_copy(x_hbm.at[srcv.at[blk]], rowbuf.at[b], gsem.at[b])

        def wait_gather(b):
            pltpu.make_async_copy(x_hbm.at[srcv.at[0]], rowbuf.at[b],
                                  gsem.at[b]).wait()

        def wait_scatters(b):
            pltpu.make_async_copy(rowbuf.at[b], S_sh.at[dstv.at[0]],
                                  ssem.at[b]).wait()
            if with_c:
                pltpu.make_async_copy(rowC.at[b], c_sh.at[srcv.at[0]],
                                      csem.at[b]).wait()

        for p in range(PF):
            gather(p, p)

        def blk_body(blk, carry):
            b = lax.rem(blk, KBUF)
            nxt = blk + PF

            @pl.when(nxt < NBLK)
            def _():
                bn = lax.rem(nxt, KBUF)

                @pl.when(nxt >= KBUF)
                def _():
                    wait_scatters(bn)

                gather(nxt, bn)

            wait_gather(b)

            def g_body(g, c2):
                wg = ewv[blk, pl.ds(g * 16, 16)]
                for j in range(16):
                    wv = jnp.full((16,), wg[j], jnp.float32)
                    e = g * 16 + j
                    for k in range(D // 16):
                        sl = pl.ds(k * 16, 16)
                        rowbuf[b, e, sl] = rowbuf[b, e, sl] * wv
                    if with_c:
                        rowC[b, e, :] = wv
                return c2

            lax.fori_loop(0, B // 16, g_body, 0)
            pltpu.async_copy(rowbuf.at[b], S_sh.at[dstv.at[blk]], ssem.at[b],
                             add=True)
            if with_c:
                pltpu.async_copy(rowC.at[b], c_sh.at[srcv.at[blk]], csem.at[b],
                                 add=True)
            return carry

        lax.fori_loop(0, NBLK, blk_body, 0)
        for p in range(KBUF):
            wait_scatters(p)
        plsc.subcore_barrier()

        for j in range(NZC):
            sl = pl.ds(sid * STRIPE + j * SRCHUNK, SRCHUNK)
            pltpu.async_copy(S_sh.at[sl], wbS.at[j], zsem)
            if with_c:
                pltpu.async_copy(c_sh.at[sl], wbc.at[j], zsem)
        for j in range(NZC):
            sl = pl.ds(sid * STRIPE + j * SRCHUNK, SRCHUNK)
            pltpu.make_async_copy(S_sh.at[sl], wbS.at[j], zsem).wait()
            pltpu.async_copy(wbS.at[j], Sp.at[cid, sl], wsem)
            if with_c:
                pltpu.make_async_copy(c_sh.at[sl], wbc.at[j], zsem).wait()
                pltpu.async_copy(wbc.at[j], cp.at[cid, sl], wsem)
        for j in range(NZC):
            sl = pl.ds(sid * STRIPE + j * SRCHUNK, SRCHUNK)
            pltpu.make_async_copy(wbS.at[j], Sp.at[cid, sl], wsem).wait()
            if with_c:
                pltpu.make_async_copy(wbc.at[j], cp.at[cid, sl], wsem).wait()

    def load_chunks(src_h, dst_h, ew_h):
        pltpu.sync_copy(src_h.at[wid], srcv)
        pltpu.sync_copy(dst_h.at[wid], dstv)
        pltpu.sync_copy(ew_h.at[wid], ewv)

    load_chunks(srcs_h, dsts_h, ews_h)
    run_phase(xs0, Ss0, csp, True)
    run_phase(xs1, Ss1, None, False)
    run_phase(xs2, Ss2, None, False)
    run_phase(xs3, Ss3, None, False)
    load_chunks(srcm_h, dstm_h, ewm_h)
    run_phase(xm, Smp, cmp, True)


_sc_edges = functools.partial(
    pl.kernel,
    _sc_body,
    out_type=[jax.ShapeDtypeStruct((NC, NP, D), jnp.float32)] * 5
    + [jax.ShapeDtypeStruct((NC, NP, CW), jnp.float32)] * 2,
    mesh=plsc.VectorSubcoreMesh(core_axis_name="c", subcore_axis_name="s"),
    compiler_params=pltpu.CompilerParams(use_tc_tiling_on_sc=False),
    scratch_types=[
        pltpu.VMEM((NBLK, B), jnp.int32),
        pltpu.VMEM((NBLK, B), jnp.int32),
        pltpu.VMEM((NBLK, B), jnp.float32),
        pltpu.VMEM((KBUF, B, D), jnp.float32),
        pltpu.VMEM((KBUF, B, CW), jnp.float32),
        pltpu.VMEM((SRCHUNK, D), jnp.float32),
        pltpu.VMEM((SRCHUNK, CW), jnp.float32),
        pltpu.VMEM((NZC, SRCHUNK, D), jnp.float32),
        pltpu.VMEM((NZC, SRCHUNK, CW), jnp.float32),
        pltpu.SemaphoreType.DMA((KBUF,)),
        pltpu.SemaphoreType.DMA((KBUF,)),
        pltpu.SemaphoreType.DMA((KBUF,)),
        pltpu.SemaphoreType.DMA,
        pltpu.SemaphoreType.DMA,
        pltpu.VMEM_SHARED((NP, D), jnp.float32),
        pltpu.VMEM_SHARED((NP, CW), jnp.float32),
    ],
)()


RB = NP // 8


def _red_body(Ss0, Ss1, Ss2, Ss3, Smp, csp, cmp, W1s, b1s, W1m, b1m,
              ts_ref, tm_ref):
    i = pl.program_id(0)

    @pl.when(i == 0)
    def _():
        ts_ref[...] = jnp.zeros_like(ts_ref)
        tm_ref[...] = jnp.zeros_like(tm_ref)

    W1s_ = W1s[...]
    hs = jnp.maximum(
        jnp.dot(Ss0[0] + Ss0[1], W1s_[0 * D:1 * D, :], preferred_element_type=jnp.float32)
        + jnp.dot(Ss1[0] + Ss1[1], W1s_[1 * D:2 * D, :], preferred_element_type=jnp.float32)
        + jnp.dot(Ss2[0] + Ss2[1], W1s_[2 * D:3 * D, :], preferred_element_type=jnp.float32)
        + jnp.dot(Ss3[0] + Ss3[1], W1s_[3 * D:4 * D, :], preferred_element_type=jnp.float32)
        + b1s[...], 0.0)
    cs = csp[0, :, 0:1] + csp[1, :, 0:1]
    ts_ref[...] += jnp.sum(cs * hs, axis=0, keepdims=True)

    hm = jnp.maximum(
        jnp.dot(Smp[0] + Smp[1], W1m[...], preferred_element_type=jnp.float32)
        + b1m[...], 0.0)
    cm = cmp[0, :, 0:1] + cmp[1, :, 0:1]
    tm_ref[...] += jnp.sum(cm * hm, axis=0, keepdims=True)


def _head_body(ts, tm, W2s, b2s, W2m, b2m, f1W, f1b, f2W, f2b, f3W, f3b,
               f4W, f4b, km, out_ref):
    inv_n = 1.0 / N
    hg_s = jnp.dot(ts[...], W2s[...], preferred_element_type=jnp.float32) * inv_n + b2s[...]
    hg_m = jnp.dot(tm[...], W2m[...], preferred_element_type=jnp.float32) * inv_n + b2m[...]

    f1 = f1W[...]
    z = (jnp.dot(hg_s, f1[0:256, :], preferred_element_type=jnp.float32)
         + jnp.dot(hg_m, f1[256:512, :], preferred_element_type=jnp.float32)
         + jnp.dot(km[...], f1[512:1150, :], preferred_element_type=jnp.float32)
         + f1b[...])
    z = jnp.maximum(z, 0.0)
    z = jnp.maximum(jnp.dot(z, f2W[...], preferred_element_type=jnp.float32) + f2b[...], 0.0)
    z = jnp.maximum(jnp.dot(z, f3W[...], preferred_element_type=jnp.float32) + f3b[...], 0.0)
    out_ref[...] = jnp.dot(z, f4W[...], preferred_element_type=jnp.float32) + f4b[...]


def kernel(feat_mol, feat_seq, edge_index_mol, edge_index_seq, edge_weight_mol,
           edge_weight_seq, smiles_kmer, W1_mol, b1_mol, W2_mol, b2_mol, W1_seq,
           b1_seq, W2_seq, b2_seq, fc1_W, fc1_b, fc2_W, fc2_b, fc3_W, fc3_b,
           fc4_W, fc4_b):
    xm = jnp.pad(feat_mol, ((0, 0), (0, D - 17)))
    W1m = jnp.pad(W1_mol, ((0, D - 17), (0, 0)))

    def chunked(a):
        a2 = a.reshape(NW, CHUNK)
        a2 = jnp.pad(a2, ((0, 0), (0, EP - CHUNK)))
        return a2.reshape(NW, NBLK, B)

    srcm = chunked(edge_index_mol[0])
    dstm = chunked(edge_index_mol[1])
    ewm = chunked(edge_weight_mol)
    srcs = chunked(edge_index_seq[0])
    dsts = chunked(edge_index_seq[1])
    ews = chunked(edge_weight_seq)

    Ss0, Ss1, Ss2, Ss3, Smp, csp, cmp = _sc_edges(
        feat_seq[:, 0:D], feat_seq[:, D:2 * D], feat_seq[:, 2 * D:3 * D],
        feat_seq[:, 3 * D:4 * D], xm,
        srcs, dsts, ews, srcm, dstm, ewm)

    s_spec = pl.BlockSpec((NC, RB, D), lambda i: (0, i, 0))
    c_spec = pl.BlockSpec((NC, RB, CW), lambda i: (0, i, 0))
    full = lambda a: pl.BlockSpec(a.shape, lambda i: tuple(0 for _ in a.shape))
    b1s2, b1m2 = b1_seq[None], b1_mol[None]
    ts, tm = pl.pallas_call(
        _red_body,
        grid=(NP // RB,),
        in_specs=[s_spec, s_spec, s_spec, s_spec, s_spec, c_spec, c_spec,
                  full(W1_seq), full(b1s2), full(W1m), full(b1m2)],
        out_specs=[pl.BlockSpec((1, 4 * D), lambda i: (0, 0)),
                   pl.BlockSpec((1, 4 * D), lambda i: (0, 0))],
        out_shape=[jax.ShapeDtypeStruct((1, 4 * D), jnp.float32),
                   jax.ShapeDtypeStruct((1, 4 * D), jnp.float32)],
    )(Ss0, Ss1, Ss2, Ss3, Smp, csp, cmp, W1_seq, b1s2, W1m, b1m2)

    out = pl.pallas_call(
        _head_body,
        out_shape=jax.ShapeDtypeStruct((1, 1), jnp.float32),
    )(ts, tm, W2_seq, b2_seq[None], W2_mol, b2_mol[None],
      fc1_W, fc1_b[None], fc2_W, fc2_b[None],
      fc3_W, fc3_b[None], fc4_W, fc4_b[None],
      smiles_kmer[None])
    return out


# speedup vs baseline: 1.3100x; 1.3100x over previous
import functools

import jax
import jax.numpy as jnp
from jax import lax
from jax.experimental import pallas as pl
from jax.experimental.pallas import tpu as pltpu
from jax.experimental.pallas import tpu_sc as plsc

N = 10000
NP = 10240
E = 320000
NC, NS = 2, 16
NW = NC * NS
CHUNK = E // NW
B = 64
EP = 10048
NBLK = EP // B
STRIPE = NP // NS
SRCHUNK = 128
NZC = STRIPE // SRCHUNK
D = 32
NQ = 4
CW = 16
KBUF = 4
PF = 2


def _sc_body(xs0, xs1, xs2, xs3, xm, srcs_h, dsts_h, ews_h, srcm_h, dstm_h, ewm_h,
             Ss0, Ss1, Ss2, Ss3, Smp, csp, cmp,
             srcv, dstv, ewv, rowbuf, rowC, zbS, zbc, wbS, wbc,
             gsem, ssem, csem, zsem, wsem,
             S_sh, c_sh):
    cid = lax.axis_index("c")
    sid = lax.axis_index("s")
    wid = cid * NS + sid
    z16 = jnp.zeros((16,), jnp.float32)

    def zrow(r, carry):
        for k in range(D // 16):
            zbS[r, pl.ds(k * 16, 16)] = z16
        zbc[r, :] = z16
        return carry

    lax.fori_loop(0, SRCHUNK, zrow, 0)

    def run_phase(x_hbm, Sp, cp, with_c):
        for j in range(NZC):
            sl = pl.ds(sid * STRIPE + j * SRCHUNK, SRCHUNK)
            pltpu.async_copy(zbS, S_sh.at[sl], zsem)
            if with_c:
                pltpu.async_copy(zbc, c_sh.at[sl], zsem)
        for j in range(NZC):
            sl = pl.ds(sid * STRIPE + j * SRCHUNK, SRCHUNK)
            pltpu.make_async_copy(zbS, S_sh.at[sl], zsem).wait()
            if with_c:
                pltpu.make_async_copy(zbc, c_sh.at[sl], zsem).wait()
        plsc.subcore_barrier()

        def gather(blk, b):
            pltpu.async_copy(x_hbm.at[srcv.at[blk]], rowbuf.at[b], gsem.at[b])

        def wait_gather(b):
            pltpu.make_async_copy(x_hbm.at[srcv.at[0]], rowbuf.at[b],
                                  gsem.at[b]).wait()

        def wait_scatters(b):
            pltpu.make_async_copy(rowbuf.at[b], S_sh.at[dstv.at[0]],
                                  ssem.at[b]).wait()
            if with_c:
                pltpu.make_async_copy(rowC.at[b], c_sh.at[srcv.at[0]],
                                      csem.at[b]).wait()

        for p in range(PF):
            gather(p, p)

        def blk_body(blk, carry):
            b = lax.rem(blk, KBUF)
            nxt = blk + PF

            @pl.when(nxt < NBLK)
            def _():
                bn = lax.rem(nxt, KBUF)

                @pl.when(nxt >= KBUF)
                def _():
                    wait_scatters(bn)

                gather(nxt, bn)

            wait_gather(b)

            def g_body(g, c2):
                wg = ewv[blk, pl.ds(g * 16, 16)]
                for j in range(16):
                    wv = jnp.full((16,), wg[j], jnp.float32)
                    e = g * 16 + j
                    for k in range(D // 16):
                        sl = pl.ds(k * 16, 16)
                        rowbuf[b, e, sl] = rowbuf[b, e, sl] * wv
                    if with_c:
                        rowC[b, e, :] = wv
                return c2

            lax.fori_loop(0, B // 16, g_body, 0)
            pltpu.async_copy(rowbuf.at[b], S_sh.at[dstv.at[blk]], ssem.at[b],
                             add=True)
            if with_c:
                pltpu.async_copy(rowC.at[b], c_sh.at[srcv.at[blk]], csem.at[b],
                                 add=True)
            return carry

        lax.fori_loop(0, NBLK, blk_body, 0)
        for p in range(KBUF):
            wait_scatters(p)
        plsc.subcore_barrier()

        for j in range(NZC):
            sl = pl.ds(sid * STRIPE + j * SRCHUNK, SRCHUNK)
            pltpu.async_copy(S_sh.at[sl], wbS.at[j], zsem)
            if with_c:
                pltpu.async_copy(c_sh.at[sl], wbc.at[j], zsem)
        for j in range(NZC):
            sl = pl.ds(sid * STRIPE + j * SRCHUNK, SRCHUNK)
            pltpu.make_async_copy(S_sh.at[sl], wbS.at[j], zsem).wait()
            pltpu.async_copy(wbS.at[j], Sp.at[cid, sl], wsem)
            if with_c:
                pltpu.make_async_copy(c_sh.at[sl], wbc.at[j], zsem).wait()
                pltpu.async_copy(wbc.at[j], cp.at[cid, sl], wsem)
        for j in range(NZC):
            sl = pl.ds(sid * STRIPE + j * SRCHUNK, SRCHUNK)
            pltpu.make_async_copy(wbS.at[j], Sp.at[cid, sl], wsem).wait()
            if with_c:
                pltpu.make_async_copy(wbc.at[j], cp.at[cid, sl], wsem).wait()

    def load_chunks(src_h, dst_h, ew_h):
        pltpu.sync_copy(src_h.at[wid], srcv)
        pltpu.sync_copy(dst_h.at[wid], dstv)
        pltpu.sync_copy(ew_h.at[wid], ewv)

    load_chunks(srcs_h, dsts_h, ews_h)
    run_phase(xs0, Ss0, csp, True)
    run_phase(xs1, Ss1, None, False)
    run_phase(xs2, Ss2, None, False)
    run_phase(xs3, Ss3, None, False)
    load_chunks(srcm_h, dstm_h, ewm_h)
    run_phase(xm, Smp, cmp, True)


_sc_edges = functools.partial(
    pl.kernel,
    _sc_body,
    out_type=[jax.ShapeDtypeStruct((NC, NP, D), jnp.float32)] * 5
    + [jax.ShapeDtypeStruct((NC, NP, CW), jnp.float32)] * 2,
    mesh=plsc.VectorSubcoreMesh(core_axis_name="c", subcore_axis_name="s"),
    compiler_params=pltpu.CompilerParams(use_tc_tiling_on_sc=False),
    scratch_types=[
        pltpu.VMEM((NBLK, B), jnp.int32),
        pltpu.VMEM((NBLK, B), jnp.int32),
        pltpu.VMEM((NBLK, B), jnp.float32),
        pltpu.VMEM((KBUF, B, D), jnp.float32),
        pltpu.VMEM((KBUF, B, CW), jnp.float32),
        pltpu.VMEM((SRCHUNK, D), jnp.float32),
        pltpu.VMEM((SRCHUNK, CW), jnp.float32),
        pltpu.VMEM((NZC, SRCHUNK, D), jnp.float32),
        pltpu.VMEM((NZC, SRCHUNK, CW), jnp.float32),
        pltpu.SemaphoreType.DMA((KBUF,)),
        pltpu.SemaphoreType.DMA((KBUF,)),
        pltpu.SemaphoreType.DMA((KBUF,)),
        pltpu.SemaphoreType.DMA,
        pltpu.SemaphoreType.DMA,
        pltpu.VMEM_SHARED((NP, D), jnp.float32),
        pltpu.VMEM_SHARED((NP, CW), jnp.float32),
    ],
)()


RB = NP // 8


def _red_body(Ss0, Ss1, Ss2, Ss3, Smp, csp, cmp, W1s, b1s, W1m, b1m,
              ts_ref, tm_ref):
    i = pl.program_id(0)

    @pl.when(i == 0)
    def _():
        ts_ref[...] = jnp.zeros_like(ts_ref)
        tm_ref[...] = jnp.zeros_like(tm_ref)

    W1s_ = W1s[...]
    hs = jnp.maximum(
        jnp.dot(Ss0[0] + Ss0[1], W1s_[0 * D:1 * D, :], preferred_element_type=jnp.float32)
        + jnp.dot(Ss1[0] + Ss1[1], W1s_[1 * D:2 * D, :], preferred_element_type=jnp.float32)
        + jnp.dot(Ss2[0] + Ss2[1], W1s_[2 * D:3 * D, :], preferred_element_type=jnp.float32)
        + jnp.dot(Ss3[0] + Ss3[1], W1s_[3 * D:4 * D, :], preferred_element_type=jnp.float32)
        + b1s[...], 0.0)
    cs = csp[0, :, 0:1] + csp[1, :, 0:1]
    ts_ref[...] += jnp.sum(cs * hs, axis=0, keepdims=True)

    hm = jnp.maximum(
        jnp.dot(Smp[0] + Smp[1], W1m[...], preferred_element_type=jnp.float32)
        + b1m[...], 0.0)
    cm = cmp[0, :, 0:1] + cmp[1, :, 0:1]
    tm_ref[...] += jnp.sum(cm * hm, axis=0, keepdims=True)


def _head_body(ts, tm, W2s, b2s, W2m, b2m, f1W, f1b, f2W, f2b, f3W, f3b,
               f4W, f4b, km, out_ref):
    inv_n = 1.0 / N
    hg_s = jnp.dot(ts[...], W2s[...], preferred_element_type=jnp.float32) * inv_n + b2s[...]
    hg_m = jnp.dot(tm[...], W2m[...], preferred_element_type=jnp.float32) * inv_n + b2m[...]

    f1 = f1W[...]
    z = (jnp.dot(hg_s, f1[0:256, :], preferred_element_type=jnp.float32)
         + jnp.dot(hg_m, f1[256:512, :], preferred_element_type=jnp.float32)
         + jnp.dot(km[...], f1[512:1150, :], preferred_element_type=jnp.float32)
         + f1b[...])
    z = jnp.maximum(z, 0.0)
    z = jnp.maximum(jnp.dot(z, f2W[...], preferred_element_type=jnp.float32) + f2b[...], 0.0)
    z = jnp.maximum(jnp.dot(z, f3W[...], preferred_element_type=jnp.float32) + f3b[...], 0.0)
    out_ref[...] = jnp.dot(z, f4W[...], preferred_element_type=jnp.float32) + f4b[...]


def kernel(feat_mol, feat_seq, edge_index_mol, edge_index_seq, edge_weight_mol,
           edge_weight_seq, smiles_kmer, W1_mol, b1_mol, W2_mol, b2_mol, W1_seq,
           b1_seq, W2_seq, b2_seq, fc1_W, fc1_b, fc2_W, fc2_b, fc3_W, fc3_b,
           fc4_W, fc4_b):
    xm = jnp.pad(feat_mol, ((0, 0), (0, D - 17)))
    W1m = jnp.pad(W1_mol, ((0, D - 17), (0, 0)))

    def chunked(a):
        a2 = a.reshape(NW, CHUNK)
        a2 = jnp.pad(a2, ((0, 0), (0, EP - CHUNK)))
        return a2.reshape(NW, NBLK, B)

    srcm = chunked(edge_index_mol[0])
    dstm = chunked(edge_index_mol[1])
    ewm = chunked(edge_weight_mol)
    srcs = chunked(edge_index_seq[0])
    dsts = chunked(edge_index_seq[1])
    ews = chunked(edge_weight_seq)

    Ss0, Ss1, Ss2, Ss3, Smp, csp, cmp = _sc_edges(
        feat_seq[:, 0:D], feat_seq[:, D:2 * D], feat_seq[:, 2 * D:3 * D],
        feat_seq[:, 3 * D:4 * D], xm,
        srcs, dsts, ews, srcm, dstm, ewm)

    s_spec = pl.BlockSpec((NC, RB, D), lambda i: (0, i, 0))
    c_spec = pl.BlockSpec((NC, RB, CW), lambda i: (0, i, 0))
    full = lambda a: pl.BlockSpec(a.shape, lambda i: tuple(0 for _ in a.shape))
    b1s2, b1m2 = b1_seq[None], b1_mol[None]
    ts, tm = pl.pallas_call(
        _red_body,
        grid=(NP // RB,),
        in_specs=[s_spec, s_spec, s_spec, s_spec, s_spec, c_spec, c_spec,
                  full(W1_seq), full(b1s2), full(W1m), full(b1m2)],
        out_specs=[pl.BlockSpec((1, 4 * D), lambda i: (0, 0)),
                   pl.BlockSpec((1, 4 * D), lambda i: (0, 0))],
        out_shape=[jax.ShapeDtypeStruct((1, 4 * D), jnp.float32),
                   jax.ShapeDtypeStruct((1, 4 * D), jnp.float32)],
    )(Ss0, Ss1, Ss2, Ss3, Smp, csp, cmp, W1_seq, b1s2, W1m, b1m2)

    out = pl.pallas_call(
        _head_body,
        out_shape=jax.ShapeDtypeStruct((1, 1), jnp.float32),
    )(ts, tm, W2_seq, b2_seq[None], W2_mol, b2_mol[None],
      fc1_W, fc1_b[None], fc2_W, fc2_b[None],
      fc3_W, fc3_b[None], fc4_W, fc4_b[None],
      smiles_kmer[None])
    return out


# speedup vs baseline: 1.6217x; 1.2380x over previous
import functools

import jax
import jax.numpy as jnp
from jax import lax
from jax.experimental import pallas as pl
from jax.experimental.pallas import tpu as pltpu
from jax.experimental.pallas import tpu_sc as plsc

N = 10000
NP = 10240
E = 320000
NC, NS = 2, 16
NW = NC * NS
CHUNK = E // NW
B = 80
EP = 10000
NBLK = EP // B
STRIPE = NP // NS
SRCHUNK = 128
NZC = STRIPE // SRCHUNK
D = 32
NQ = 4
CW = 16
KBUF = 4
PF = 2


def _sc_body(xs0, xs1, xs2, xs3, xm, srcs_h, dsts_h, ews_h, srcm_h, dstm_h, ewm_h,
             Ss0, Ss1, Ss2, Ss3, Smp, csp, cmp,
             srcv, dstv, ewv, rowbuf, rowC, zbS, zbc, wbS, wbc,
             gsem, ssem, csem, zsem, wsem,
             S_sh, c_sh):
    cid = lax.axis_index("c")
    sid = lax.axis_index("s")
    wid = cid * NS + sid
    z16 = jnp.zeros((16,), jnp.float32)

    def zrow(r, carry):
        for k in range(D // 16):
            zbS[r, pl.ds(k * 16, 16)] = z16
        zbc[r, :] = z16
        return carry

    lax.fori_loop(0, SRCHUNK, zrow, 0)

    def run_phase(x_hbm, Sp, cp, with_c):
        for j in range(NZC):
            sl = pl.ds(sid * STRIPE + j * SRCHUNK, SRCHUNK)
            pltpu.async_copy(zbS, S_sh.at[sl], zsem)
            if with_c:
                pltpu.async_copy(zbc, c_sh.at[sl], zsem)
        for j in range(NZC):
            sl = pl.ds(sid * STRIPE + j * SRCHUNK, SRCHUNK)
            pltpu.make_async_copy(zbS, S_sh.at[sl], zsem).wait()
            if with_c:
                pltpu.make_async_copy(zbc, c_sh.at[sl], zsem).wait()
        plsc.subcore_barrier()

        def gather(blk, b):
            pltpu.async_copy(x_hbm.at[srcv.at[blk]], rowbuf.at[b], gsem.at[b])

        def wait_gather(b):
            pltpu.make_async_copy(x_hbm.at[srcv.at[0]], rowbuf.at[b],
                                  gsem.at[b]).wait()

        def wait_scatters(b):
            pltpu.make_async_copy(rowbuf.at[b], S_sh.at[dstv.at[0]],
                                  ssem.at[b]).wait()
            if with_c:
                pltpu.make_async_copy(rowC.at[b], c_sh.at[srcv.at[0]],
                                      csem.at[b]).wait()

        for p in range(PF):
            gather(p, p)

        def blk_body(blk, carry):
            b = lax.rem(blk, KBUF)
            nxt = blk + PF

            @pl.when(nxt < NBLK)
            def _():
                bn = lax.rem(nxt, KBUF)

                @pl.when(nxt >= KBUF)
                def _():
                    wait_scatters(bn)

                gather(nxt, bn)

            wait_gather(b)

            def g_body(g, c2):
                wg = ewv[blk, pl.ds(g * 16, 16)]
                for j in range(16):
                    wv = jnp.full((16,), wg[j], jnp.float32)
                    e = g * 16 + j
                    for k in range(D // 16):
                        sl = pl.ds(k * 16, 16)
                        rowbuf[b, e, sl] = rowbuf[b, e, sl] * wv
                    if with_c:
                        rowC[b, e, :] = wv
                return c2

            lax.fori_loop(0, B // 16, g_body, 0)
            pltpu.async_copy(rowbuf.at[b], S_sh.at[dstv.at[blk]], ssem.at[b],
                             add=True)
            if with_c:
                pltpu.async_copy(rowC.at[b], c_sh.at[srcv.at[blk]], csem.at[b],
                                 add=True)
            return carry

        lax.fori_loop(0, NBLK, blk_body, 0)
        for p in range(KBUF):
            wait_scatters(p)
        plsc.subcore_barrier()

        for j in range(NZC):
            sl = pl.ds(sid * STRIPE + j * SRCHUNK, SRCHUNK)
            pltpu.async_copy(S_sh.at[sl], wbS.at[j], zsem)
            if with_c:
                pltpu.async_copy(c_sh.at[sl], wbc.at[j], zsem)
        for j in range(NZC):
            sl = pl.ds(sid * STRIPE + j * SRCHUNK, SRCHUNK)
            pltpu.make_async_copy(S_sh.at[sl], wbS.at[j], zsem).wait()
            pltpu.async_copy(wbS.at[j], Sp.at[cid, sl], wsem)
            if with_c:
                pltpu.make_async_copy(c_sh.at[sl], wbc.at[j], zsem).wait()
                pltpu.async_copy(wbc.at[j], cp.at[cid, sl], wsem)
        for j in range(NZC):
            sl = pl.ds(sid * STRIPE + j * SRCHUNK, SRCHUNK)
            pltpu.make_async_copy(wbS.at[j], Sp.at[cid, sl], wsem).wait()
            if with_c:
                pltpu.make_async_copy(wbc.at[j], cp.at[cid, sl], wsem).wait()

    def load_chunks(src_h, dst_h, ew_h):
        pltpu.sync_copy(src_h.at[wid], srcv)
        pltpu.sync_copy(dst_h.at[wid], dstv)
        pltpu.sync_copy(ew_h.at[wid], ewv)

    load_chunks(srcs_h, dsts_h, ews_h)
    run_phase(xs0, Ss0, csp, True)
    run_phase(xs1, Ss1, None, False)
    run_phase(xs2, Ss2, None, False)
    run_phase(xs3, Ss3, None, False)
    load_chunks(srcm_h, dstm_h, ewm_h)
    run_phase(xm, Smp, cmp, True)


_sc_edges = functools.partial(
    pl.kernel,
    _sc_body,
    out_type=[jax.ShapeDtypeStruct((NC, NP, D), jnp.float32)] * 5
    + [jax.ShapeDtypeStruct((NC, NP, CW), jnp.float32)] * 2,
    mesh=plsc.VectorSubcoreMesh(core_axis_name="c", subcore_axis_name="s"),
    compiler_params=pltpu.CompilerParams(use_tc_tiling_on_sc=False),
    scratch_types=[
        pltpu.VMEM((NBLK, B), jnp.int32),
        pltpu.VMEM((NBLK, B), jnp.int32),
        pltpu.VMEM((NBLK, B), jnp.float32),
        pltpu.VMEM((KBUF, B, D), jnp.float32),
        pltpu.VMEM((KBUF, B, CW), jnp.float32),
        pltpu.VMEM((SRCHUNK, D), jnp.float32),
        pltpu.VMEM((SRCHUNK, CW), jnp.float32),
        pltpu.VMEM((NZC, SRCHUNK, D), jnp.float32),
        pltpu.VMEM((NZC, SRCHUNK, CW), jnp.float32),
        pltpu.SemaphoreType.DMA((KBUF,)),
        pltpu.SemaphoreType.DMA((KBUF,)),
        pltpu.SemaphoreType.DMA((KBUF,)),
        pltpu.SemaphoreType.DMA,
        pltpu.SemaphoreType.DMA,
        pltpu.VMEM_SHARED((NP, D), jnp.float32),
        pltpu.VMEM_SHARED((NP, CW), jnp.float32),
    ],
)()


RB = NP // 8


def _red_body(Ss0, Ss1, Ss2, Ss3, Smp, csp, cmp, W1s, b1s, W1m, b1m,
              ts_ref, tm_ref):
    i = pl.program_id(0)

    @pl.when(i == 0)
    def _():
        ts_ref[...] = jnp.zeros_like(ts_ref)
        tm_ref[...] = jnp.zeros_like(tm_ref)

    W1s_ = W1s[...]
    hs = jnp.maximum(
        jnp.dot(Ss0[0] + Ss0[1], W1s_[0 * D:1 * D, :], preferred_element_type=jnp.float32)
        + jnp.dot(Ss1[0] + Ss1[1], W1s_[1 * D:2 * D, :], preferred_element_type=jnp.float32)
        + jnp.dot(Ss2[0] + Ss2[1], W1s_[2 * D:3 * D, :], preferred_element_type=jnp.float32)
        + jnp.dot(Ss3[0] + Ss3[1], W1s_[3 * D:4 * D, :], preferred_element_type=jnp.float32)
        + b1s[...], 0.0)
    cs = csp[0, :, 0:1] + csp[1, :, 0:1]
    ts_ref[...] += jnp.sum(cs * hs, axis=0, keepdims=True)

    hm = jnp.maximum(
        jnp.dot(Smp[0] + Smp[1], W1m[...], preferred_element_type=jnp.float32)
        + b1m[...], 0.0)
    cm = cmp[0, :, 0:1] + cmp[1, :, 0:1]
    tm_ref[...] += jnp.sum(cm * hm, axis=0, keepdims=True)


def _head_body(ts, tm, W2s, b2s, W2m, b2m, f1W, f1b, f2W, f2b, f3W, f3b,
               f4W, f4b, km, out_ref):
    inv_n = 1.0 / N
    hg_s = jnp.dot(ts[...], W2s[...], preferred_element_type=jnp.float32) * inv_n + b2s[...]
    hg_m = jnp.dot(tm[...], W2m[...], preferred_element_type=jnp.float32) * inv_n + b2m[...]

    f1 = f1W[...]
    z = (jnp.dot(hg_s, f1[0:256, :], preferred_element_type=jnp.float32)
         + jnp.dot(hg_m, f1[256:512, :], preferred_element_type=jnp.float32)
         + jnp.dot(km[...], f1[512:1150, :], preferred_element_type=jnp.float32)
         + f1b[...])
    z = jnp.maximum(z, 0.0)
    z = jnp.maximum(jnp.dot(z, f2W[...], preferred_element_type=jnp.float32) + f2b[...], 0.0)
    z = jnp.maximum(jnp.dot(z, f3W[...], preferred_element_type=jnp.float32) + f3b[...], 0.0)
    out_ref[...] = jnp.dot(z, f4W[...], preferred_element_type=jnp.float32) + f4b[...]


def kernel(feat_mol, feat_seq, edge_index_mol, edge_index_seq, edge_weight_mol,
           edge_weight_seq, smiles_kmer, W1_mol, b1_mol, W2_mol, b2_mol, W1_seq,
           b1_seq, W2_seq, b2_seq, fc1_W, fc1_b, fc2_W, fc2_b, fc3_W, fc3_b,
           fc4_W, fc4_b):
    xm = jnp.pad(feat_mol, ((0, 0), (0, D - 17)))
    W1m = jnp.pad(W1_mol, ((0, D - 17), (0, 0)))

    def chunked(a):
        a2 = a.reshape(NW, CHUNK)
        a2 = jnp.pad(a2, ((0, 0), (0, EP - CHUNK)))
        return a2.reshape(NW, NBLK, B)

    srcm = chunked(edge_index_mol[0])
    dstm = chunked(edge_index_mol[1])
    ewm = chunked(edge_weight_mol)
    srcs = chunked(edge_index_seq[0])
    dsts = chunked(edge_index_seq[1])
    ews = chunked(edge_weight_seq)

    Ss0, Ss1, Ss2, Ss3, Smp, csp, cmp = _sc_edges(
        feat_seq[:, 0:D], feat_seq[:, D:2 * D], feat_seq[:, 2 * D:3 * D],
        feat_seq[:, 3 * D:4 * D], xm,
        srcs, dsts, ews, srcm, dstm, ewm)

    s_spec = pl.BlockSpec((NC, RB, D), lambda i: (0, i, 0))
    c_spec = pl.BlockSpec((NC, RB, CW), lambda i: (0, i, 0))
    full = lambda a: pl.BlockSpec(a.shape, lambda i: tuple(0 for _ in a.shape))
    b1s2, b1m2 = b1_seq[None], b1_mol[None]
    ts, tm = pl.pallas_call(
        _red_body,
        grid=(NP // RB,),
        in_specs=[s_spec, s_spec, s_spec, s_spec, s_spec, c_spec, c_spec,
                  full(W1_seq), full(b1s2), full(W1m), full(b1m2)],
        out_specs=[pl.BlockSpec((1, 4 * D), lambda i: (0, 0)),
                   pl.BlockSpec((1, 4 * D), lambda i: (0, 0))],
        out_shape=[jax.ShapeDtypeStruct((1, 4 * D), jnp.float32),
                   jax.ShapeDtypeStruct((1, 4 * D), jnp.float32)],
    )(Ss0, Ss1, Ss2, Ss3, Smp, csp, cmp, W1_seq, b1s2, W1m, b1m2)

    out = pl.pallas_call(
        _head_body,
        out_shape=jax.ShapeDtypeStruct((1, 1), jnp.float32),
    )(ts, tm, W2_seq, b2_seq[None], W2_mol, b2_mol[None],
      fc1_W, fc1_b[None], fc2_W, fc2_b[None],
      fc3_W, fc3_b[None], fc4_W, fc4_b[None],
      smiles_kmer[None])
    return out
